# Initial kernel scaffold; baseline (speedup 1.0000x reference)
import jax
import jax.numpy as jnp
from jax.experimental import pallas as pl


def kernel(x, edge_index, edge_type, R_diagonal):
    raise NotImplementedError("write your pallas kernel here")


# speedup vs baseline: 1.1272x; 1.1272x over previous
import functools

import jax
import jax.numpy as jnp
from jax import lax
from jax.experimental import pallas as pl
from jax.experimental.pallas import tpu as pltpu
from jax.experimental.pallas import tpu_sc as plsc

N_NODES = 10000
N_EDGES = 320000
D = 128
N_REL = 64

NC = 2
NS = 16
NW = NC * NS
EDGES_PER_WORKER = N_EDGES // NW
C = 80
NCHUNK = EDGES_PER_WORKER // C
DSTEP = D // 16


def _make_sc_kernel():
    mesh = plsc.VectorSubcoreMesh(core_axis_name="c", subcore_axis_name="s")

    @functools.partial(
        pl.kernel,
        out_type=jax.ShapeDtypeStruct((N_EDGES,), jnp.float32),
        mesh=mesh,
        compiler_params=pltpu.CompilerParams(needs_layout_passes=False),
        scratch_types=[
            pltpu.VMEM((C,), jnp.int32),
            pltpu.VMEM((C,), jnp.int32),
            pltpu.VMEM((C,), jnp.int32),
            pltpu.VMEM((C, D), jnp.float32),
            pltpu.VMEM((C, D), jnp.float32),
            pltpu.VMEM((N_REL, D), jnp.float32),
            pltpu.VMEM((C,), jnp.float32),
            pltpu.SemaphoreType.DMA,
            pltpu.SemaphoreType.DMA,
        ],
    )
    def dist_mult(src_hbm, dst_hbm, et_hbm, x_hbm, r_hbm, out_hbm,
                  idx_s, idx_o, etv, srows, orows, rloc, outv, sem_s, sem_o):
        wid = lax.axis_index("s") * NC + lax.axis_index("c")
        base0 = wid * EDGES_PER_WORKER
        pltpu.sync_copy(r_hbm, rloc)

        def chunk_body(ci, carry):
            base = base0 + ci * C
            pltpu.sync_copy(src_hbm.at[pl.ds(base, C)], idx_s)
            pltpu.sync_copy(dst_hbm.at[pl.ds(base, C)], idx_o)
            pltpu.sync_copy(et_hbm.at[pl.ds(base, C)], etv)
            cp_s = pltpu.async_copy(x_hbm.at[idx_s], srows, sem_s)
            cp_o = pltpu.async_copy(x_hbm.at[idx_o], orows, sem_o)
            cp_s.wait()
            cp_o.wait()

            lane = lax.iota(jnp.int32, 16)

            def group_body(g, carry2):
                eids = g * 16 + lane
                et16 = etv[pl.ds(g * 16, 16)]

                def d_body(d, acc):
                    dcol = jnp.full((16,), d, jnp.int32)
                    sv = plsc.load_gather(srows, [eids, dcol])
                    ov = plsc.load_gather(orows, [eids, dcol])
                    rv = plsc.load_gather(rloc, [et16, dcol])
                    return acc + sv * ov * rv

                scores = lax.fori_loop(
                    0, D, d_body, jnp.zeros((16,), jnp.float32), unroll=8)
                outv[pl.ds(g * 16, 16)] = scores
                return carry2

            lax.fori_loop(0, C // 16, group_body, 0)
            pltpu.sync_copy(outv, out_hbm.at[pl.ds(base, C)])
            return carry

        lax.fori_loop(0, NCHUNK, chunk_body, 0)

    return dist_mult


_dist_mult_sc = _make_sc_kernel()


@jax.jit
def kernel(x, edge_index, edge_type, R_diagonal):
    src = edge_index[0, :].astype(jnp.int32)
    dst = edge_index[1, :].astype(jnp.int32)
    et = edge_type.astype(jnp.int32)
    return _dist_mult_sc(src, dst, et, x, R_diagonal)


# speedup vs baseline: 1.3342x; 1.1836x over previous
import functools

import jax
import jax.numpy as jnp
from jax import lax
from jax.experimental import pallas as pl
from jax.experimental.pallas import tpu as pltpu
from jax.experimental.pallas import tpu_sc as plsc

N_NODES = 10000
N_EDGES = 320000
D = 128
N_REL = 64

NC = 2
NS = 16
NW = NC * NS
EPW = N_EDGES // NW
C = 80
NCHUNK = EPW // C
NPAIR = NCHUNK // 2


def _make_sc_kernel():
    mesh = plsc.VectorSubcoreMesh(core_axis_name="c", subcore_axis_name="s")

    @functools.partial(
        pl.kernel,
        out_type=jax.ShapeDtypeStruct((N_EDGES,), jnp.float32),
        mesh=mesh,
        compiler_params=pltpu.CompilerParams(needs_layout_passes=False),
        scratch_types=[
            pltpu.VMEM((EPW,), jnp.int32),
            pltpu.VMEM((EPW,), jnp.int32),
            pltpu.VMEM((EPW,), jnp.int32),
            pltpu.VMEM((EPW,), jnp.float32),
            pltpu.VMEM((N_REL * D,), jnp.float32),
            pltpu.VMEM((C, D), jnp.float32),
            pltpu.VMEM((C, D), jnp.float32),
            pltpu.VMEM((C, D), jnp.float32),
            pltpu.VMEM((C, D), jnp.float32),
            pltpu.SemaphoreType.DMA,
            pltpu.SemaphoreType.DMA,
            pltpu.SemaphoreType.DMA,
            pltpu.SemaphoreType.DMA,
        ],
    )
    def dist_mult(src_hbm, dst_hbm, et_hbm, x_hbm, r_hbm, out_hbm,
                  idx_s, idx_o, et_all, out_all, rflat,
                  sr0, sr1, or0, or1, sem_s0, sem_s1, sem_o0, sem_o1):
        wid = lax.axis_index("s") * NC + lax.axis_index("c")
        base0 = wid * EPW
        pltpu.sync_copy(r_hbm, rflat)
        pltpu.sync_copy(src_hbm.at[pl.ds(base0, EPW)], idx_s)
        pltpu.sync_copy(dst_hbm.at[pl.ds(base0, EPW)], idx_o)
        pltpu.sync_copy(et_hbm.at[pl.ds(base0, EPW)], et_all)

        lane = lax.iota(jnp.int32, 16)

        def issue(c, srows, orows, sem_s, sem_o):
            pltpu.async_copy(x_hbm.at[idx_s.at[pl.ds(c * C, C)]], srows, sem_s)
            pltpu.async_copy(x_hbm.at[idx_o.at[pl.ds(c * C, C)]], orows, sem_o)

        def wait(c, srows, orows, sem_s, sem_o):
            pltpu.make_async_copy(
                x_hbm.at[idx_s.at[pl.ds(c * C, C)]], srows, sem_s).wait()
            pltpu.make_async_copy(
                x_hbm.at[idx_o.at[pl.ds(c * C, C)]], orows, sem_o).wait()

        def compute(c, srows, orows):
            base = c * C
            for g in range(C // 16):
                et16 = et_all[pl.ds(base + g * 16, 16)]
                roff = et16 * D
                eids = g * 16 + lane

                def d_body(d, acc):
                    dcol = jnp.full((16,), d, jnp.int32)
                    sv = plsc.load_gather(srows, [eids, dcol])
                    ov = plsc.load_gather(orows, [eids, dcol])
                    rv = plsc.load_gather(rflat, [roff + d])
                    return acc + sv * ov * rv

                scores = lax.fori_loop(
                    0, D, d_body, jnp.zeros((16,), jnp.float32), unroll=8)
                out_all[pl.ds(base + g * 16, 16)] = scores

        issue(0, sr0, or0, sem_s0, sem_o0)
        issue(1, sr1, or1, sem_s1, sem_o1)

        def pair_body(i, carry):
            c0 = 2 * i
            c1 = 2 * i + 1
            wait(c0, sr0, or0, sem_s0, sem_o0)
            compute(c0, sr0, or0)

            @pl.when(c0 + 2 < NCHUNK)
            def _():
                issue(c0 + 2, sr0, or0, sem_s0, sem_o0)

            wait(c1, sr1, or1, sem_s1, sem_o1)
            compute(c1, sr1, or1)

            @pl.when(c1 + 2 < NCHUNK)
            def _():
                issue(c1 + 2, sr1, or1, sem_s1, sem_o1)

            return carry

        lax.fori_loop(0, NPAIR, pair_body, 0)

        if NCHUNK % 2 == 1:
            last = NCHUNK - 1
            wait(last, sr0, or0, sem_s0, sem_o0)
            compute(last, sr0, or0)

        pltpu.sync_copy(out_all, out_hbm.at[pl.ds(base0, EPW)])

    return dist_mult


_dist_mult_sc = _make_sc_kernel()


@jax.jit
def kernel(x, edge_index, edge_type, R_diagonal):
    src = edge_index[0, :].astype(jnp.int32)
    dst = edge_index[1, :].astype(jnp.int32)
    et = edge_type.astype(jnp.int32)
    return _dist_mult_sc(src, dst, et, x, R_diagonal.reshape(-1))


# speedup vs baseline: 11.5694x; 8.6717x over previous
import functools

import jax
import jax.numpy as jnp
from jax import lax
from jax.experimental import pallas as pl
from jax.experimental.pallas import tpu as pltpu
from jax.experimental.pallas import tpu_sc as plsc

N_NODES = 10000
N_EDGES = 320000
D = 128
N_REL = 64

NC = 2
NS = 16
NW = NC * NS
EPW = N_EDGES // NW
C = 80
NCHUNK = EPW // C
NPAIR = NCHUNK // 2


def _make_sc_kernel():
    mesh = plsc.VectorSubcoreMesh(core_axis_name="c", subcore_axis_name="s")

    @functools.partial(
        pl.kernel,
        out_type=jax.ShapeDtypeStruct((N_EDGES,), jnp.float32),
        mesh=mesh,
        compiler_params=pltpu.CompilerParams(needs_layout_passes=False),
        scratch_types=[
            pltpu.VMEM((EPW,), jnp.int32),
            pltpu.VMEM((EPW,), jnp.int32),
            pltpu.VMEM((EPW,), jnp.int32),
            pltpu.VMEM((EPW,), jnp.float32),
            pltpu.VMEM((N_REL * D,), jnp.float32),
            pltpu.VMEM((C, D), jnp.float32),
            pltpu.VMEM((C, D), jnp.float32),
            pltpu.VMEM((C, D), jnp.float32),
            pltpu.VMEM((C, D), jnp.float32),
            pltpu.SemaphoreType.DMA,
            pltpu.SemaphoreType.DMA,
            pltpu.SemaphoreType.DMA,
            pltpu.SemaphoreType.DMA,
        ],
    )
    def dist_mult(src_hbm, dst_hbm, et_hbm, x_hbm, r_hbm, out_hbm,
                  idx_s, idx_o, et_all, out_all, rflat,
                  sr0, sr1, or0, or1, sem_s0, sem_s1, sem_o0, sem_o1):
        wid = lax.axis_index("s") * NC + lax.axis_index("c")
        base0 = wid * EPW
        pltpu.sync_copy(r_hbm, rflat)
        pltpu.sync_copy(src_hbm.at[pl.ds(base0, EPW)], idx_s)
        pltpu.sync_copy(dst_hbm.at[pl.ds(base0, EPW)], idx_o)
        pltpu.sync_copy(et_hbm.at[pl.ds(base0, EPW)], et_all)

        lane = lax.iota(jnp.int32, 16)

        def issue(c, srows, orows, sem_s, sem_o):
            pltpu.async_copy(x_hbm.at[idx_s.at[pl.ds(c * C, C)]], srows, sem_s)
            pltpu.async_copy(x_hbm.at[idx_o.at[pl.ds(c * C, C)]], orows, sem_o)

        def wait(c, srows, orows, sem_s, sem_o):
            pltpu.make_async_copy(
                x_hbm.at[idx_s.at[pl.ds(c * C, C)]], srows, sem_s).wait()
            pltpu.make_async_copy(
                x_hbm.at[idx_o.at[pl.ds(c * C, C)]], orows, sem_o).wait()

        def compute(c, srows, orows):
            base = c * C
            for g in range(C // 16):
                et16 = et_all[pl.ds(base + g * 16, 16)]
                roff = et16 * D
                eids = g * 16 + lane

                def d_body(d, acc):
                    dcol = (lane + d) & (D - 1)
                    sv = plsc.load_gather(srows, [eids, dcol])
                    ov = plsc.load_gather(orows, [eids, dcol])
                    rv = plsc.load_gather(rflat, [roff + dcol])
                    return acc + sv * ov * rv

                scores = lax.fori_loop(
                    0, D, d_body, jnp.zeros((16,), jnp.float32), unroll=8)
                out_all[pl.ds(base + g * 16, 16)] = scores

        issue(0, sr0, or0, sem_s0, sem_o0)
        issue(1, sr1, or1, sem_s1, sem_o1)

        def pair_body(i, carry):
            c0 = 2 * i
            c1 = 2 * i + 1
            wait(c0, sr0, or0, sem_s0, sem_o0)
            compute(c0, sr0, or0)

            @pl.when(c0 + 2 < NCHUNK)
            def _():
                issue(c0 + 2, sr0, or0, sem_s0, sem_o0)

            wait(c1, sr1, or1, sem_s1, sem_o1)
            compute(c1, sr1, or1)

            @pl.when(c1 + 2 < NCHUNK)
            def _():
                issue(c1 + 2, sr1, or1, sem_s1, sem_o1)

            return carry

        lax.fori_loop(0, NPAIR, pair_body, 0)

        if NCHUNK % 2 == 1:
            last = NCHUNK - 1
            wait(last, sr0, or0, sem_s0, sem_o0)
            compute(last, sr0, or0)

        pltpu.sync_copy(out_all, out_hbm.at[pl.ds(base0, EPW)])

    return dist_mult


_dist_mult_sc = _make_sc_kernel()


@jax.jit
def kernel(x, edge_index, edge_type, R_diagonal):
    src = edge_index[0, :].astype(jnp.int32)
    dst = edge_index[1, :].astype(jnp.int32)
    et = edge_type.astype(jnp.int32)
    return _dist_mult_sc(src, dst, et, x, R_diagonal.reshape(-1))


# speedup vs baseline: 13.1199x; 1.1340x over previous
import functools

import jax
import jax.numpy as jnp
from jax import lax
from jax.experimental import pallas as pl
from jax.experimental.pallas import tpu as pltpu
from jax.experimental.pallas import tpu_sc as plsc

N_NODES = 10000
N_EDGES = 320000
D = 128
N_REL = 64

NC = 2
NS = 16
NW = NC * NS
EPW = N_EDGES // NW
C = 80
NCHUNK = EPW // C
NPAIR = NCHUNK // 2
W = D // 2


def _make_sc_kernel():
    mesh = plsc.VectorSubcoreMesh(core_axis_name="c", subcore_axis_name="s")

    @functools.partial(
        pl.kernel,
        out_type=jax.ShapeDtypeStruct((N_EDGES,), jnp.float32),
        mesh=mesh,
        compiler_params=pltpu.CompilerParams(
            needs_layout_passes=False, use_tc_tiling_on_sc=False),
        scratch_types=[
            pltpu.VMEM((EPW,), jnp.int32),
            pltpu.VMEM((EPW,), jnp.int32),
            pltpu.VMEM((EPW,), jnp.int32),
            pltpu.VMEM((EPW,), jnp.float32),
            pltpu.VMEM((N_REL * W,), jnp.int32),
            pltpu.VMEM((C, W), jnp.int32),
            pltpu.VMEM((C, W), jnp.int32),
            pltpu.VMEM((C, W), jnp.int32),
            pltpu.VMEM((C, W), jnp.int32),
            pltpu.SemaphoreType.DMA,
            pltpu.SemaphoreType.DMA,
            pltpu.SemaphoreType.DMA,
            pltpu.SemaphoreType.DMA,
        ],
    )
    def dist_mult(src_hbm, dst_hbm, et_hbm, x_hbm, r_hbm, out_hbm,
                  idx_s, idx_o, et_all, out_all, rflat,
                  sr0, sr1, or0, or1, sem_s0, sem_s1, sem_o0, sem_o1):
        wid = lax.axis_index("s") * NC + lax.axis_index("c")
        base0 = wid * EPW
        pltpu.sync_copy(r_hbm, rflat)
        pltpu.sync_copy(src_hbm.at[pl.ds(base0, EPW)], idx_s)
        pltpu.sync_copy(dst_hbm.at[pl.ds(base0, EPW)], idx_o)
        pltpu.sync_copy(et_hbm.at[pl.ds(base0, EPW)], et_all)

        lane = lax.iota(jnp.int32, 16)

        def issue(c, srows, orows, sem_s, sem_o):
            pltpu.async_copy(x_hbm.at[idx_s.at[pl.ds(c * C, C)]], srows, sem_s)
            pltpu.async_copy(x_hbm.at[idx_o.at[pl.ds(c * C, C)]], orows, sem_o)

        def wait(c, srows, orows, sem_s, sem_o):
            pltpu.make_async_copy(
                x_hbm.at[idx_s.at[pl.ds(c * C, C)]], srows, sem_s).wait()
            pltpu.make_async_copy(
                x_hbm.at[idx_o.at[pl.ds(c * C, C)]], orows, sem_o).wait()

        def compute(c, srows, orows):
            base = c * C
            for g in range(C // 16):
                et16 = et_all[pl.ds(base + g * 16, 16)]
                roff = et16 * W
                eids = g * 16 + lane

                def d_body(j, acc):
                    wcol = (lane + j) & (W - 1)
                    sw = plsc.load_gather(srows, [eids, wcol])
                    ow = plsc.load_gather(orows, [eids, wcol])
                    rw = plsc.load_gather(rflat, [roff + wcol])
                    sv = plsc.bitcast(sw, jnp.bfloat16)
                    ov = plsc.bitcast(ow, jnp.bfloat16)
                    rv = plsc.bitcast(rw, jnp.bfloat16)
                    prod = sv * ov * rv
                    pa, pb = plsc.unpack(
                        prod, format=plsc.PackFormat.INTERLEAVED)
                    return acc + pa + pb

                scores = lax.fori_loop(
                    0, W, d_body, jnp.zeros((16,), jnp.float32), unroll=8)
                out_all[pl.ds(base + g * 16, 16)] = scores

        issue(0, sr0, or0, sem_s0, sem_o0)
        issue(1, sr1, or1, sem_s1, sem_o1)

        def pair_body(i, carry):
            c0 = 2 * i
            c1 = 2 * i + 1
            wait(c0, sr0, or0, sem_s0, sem_o0)
            compute(c0, sr0, or0)

            @pl.when(c0 + 2 < NCHUNK)
            def _():
                issue(c0 + 2, sr0, or0, sem_s0, sem_o0)

            wait(c1, sr1, or1, sem_s1, sem_o1)
            compute(c1, sr1, or1)

            @pl.when(c1 + 2 < NCHUNK)
            def _():
                issue(c1 + 2, sr1, or1, sem_s1, sem_o1)

            return carry

        lax.fori_loop(0, NPAIR, pair_body, 0)

        if NCHUNK % 2 == 1:
            last = NCHUNK - 1
            wait(last, sr0, or0, sem_s0, sem_o0)
            compute(last, sr0, or0)

        pltpu.sync_copy(out_all, out_hbm.at[pl.ds(base0, EPW)])

    return dist_mult


_dist_mult_sc = _make_sc_kernel()


def _pack_bf16(a):
    b = a.astype(jnp.bfloat16)
    return jax.lax.bitcast_convert_type(
        b.reshape(*a.shape[:-1], a.shape[-1] // 2, 2), jnp.int32)


@jax.jit
def kernel(x, edge_index, edge_type, R_diagonal):
    src = edge_index[0, :].astype(jnp.int32)
    dst = edge_index[1, :].astype(jnp.int32)
    et = edge_type.astype(jnp.int32)
    xp = _pack_bf16(x)
    rp = _pack_bf16(R_diagonal).reshape(-1)
    return _dist_mult_sc(src, dst, et, xp, rp)


# speedup vs baseline: 16.8299x; 1.2828x over previous
import functools

import jax
import jax.numpy as jnp
from jax import lax
from jax.experimental import pallas as pl
from jax.experimental.pallas import tpu as pltpu
from jax.experimental.pallas import tpu_sc as plsc

N_NODES = 10000
N_EDGES = 320000
D = 128
N_REL = 64

NC = 2
NS = 16
NW = NC * NS
EPW = N_EDGES // NW
C = 80
NCHUNK = EPW // C
NPAIR = NCHUNK // 2
W = D // 2


def _pack_tc_kernel(x_ref, out_ref):
    u = pltpu.bitcast(x_ref[...], jnp.uint32)
    t = u + jnp.uint32(0x7FFF) + ((u >> 16) & jnp.uint32(1))
    lo = t[:, :W]
    hi = t[:, W:]
    packed = (hi & jnp.uint32(0xFFFF0000)) | (lo >> 16)
    out_ref[...] = pltpu.bitcast(packed, jnp.int32)


def _pack_x(x):
    return pl.pallas_call(
        _pack_tc_kernel,
        out_shape=jax.ShapeDtypeStruct((N_NODES, W), jnp.int32),
    )(x)


def _make_sc_kernel():
    mesh = plsc.VectorSubcoreMesh(core_axis_name="c", subcore_axis_name="s")

    @functools.partial(
        pl.kernel,
        out_type=jax.ShapeDtypeStruct((N_EDGES,), jnp.float32),
        mesh=mesh,
        compiler_params=pltpu.CompilerParams(
            needs_layout_passes=False, use_tc_tiling_on_sc=False),
        scratch_types=[
            pltpu.VMEM((EPW,), jnp.int32),
            pltpu.VMEM((EPW,), jnp.int32),
            pltpu.VMEM((EPW,), jnp.int32),
            pltpu.VMEM((EPW,), jnp.float32),
            pltpu.VMEM((N_REL, D), jnp.float32),
            pltpu.VMEM((N_REL * W,), jnp.int32),
            pltpu.VMEM((C, W), jnp.int32),
            pltpu.VMEM((C, W), jnp.int32),
            pltpu.VMEM((C, W), jnp.int32),
            pltpu.VMEM((C, W), jnp.int32),
            pltpu.SemaphoreType.DMA,
            pltpu.SemaphoreType.DMA,
            pltpu.SemaphoreType.DMA,
            pltpu.SemaphoreType.DMA,
        ],
    )
    def dist_mult(ei_hbm, et_hbm, x_hbm, r_hbm, out_hbm,
                  idx_s, idx_o, et_all, out_all, rtmp, rflat,
                  sr0, sr1, or0, or1, sem_s0, sem_s1, sem_o0, sem_o1):
        wid = lax.axis_index("s") * NC + lax.axis_index("c")
        base0 = wid * EPW
        pltpu.sync_copy(r_hbm, rtmp)
        pltpu.sync_copy(ei_hbm.at[0, pl.ds(base0, EPW)], idx_s)
        pltpu.sync_copy(ei_hbm.at[1, pl.ds(base0, EPW)], idx_o)
        pltpu.sync_copy(et_hbm.at[pl.ds(base0, EPW)], et_all)

        lane = lax.iota(jnp.int32, 16)

        def rpack_body(t, carry):
            for j in range(W // 16):
                a = rtmp[t, pl.ds(16 * j, 16)]
                b = rtmp[t, pl.ds(16 * j + W, 16)]
                word = plsc.bitcast(
                    plsc.pack(a, b, format=plsc.PackFormat.INTERLEAVED),
                    jnp.int32)
                rflat[pl.ds(t * W + 16 * j, 16)] = word
            return carry

        lax.fori_loop(0, N_REL, rpack_body, 0)

        def issue(c, srows, orows, sem_s, sem_o):
            pltpu.async_copy(x_hbm.at[idx_s.at[pl.ds(c * C, C)]], srows, sem_s)
            pltpu.async_copy(x_hbm.at[idx_o.at[pl.ds(c * C, C)]], orows, sem_o)

        def wait(c, srows, orows, sem_s, sem_o):
            pltpu.make_async_copy(
                x_hbm.at[idx_s.at[pl.ds(c * C, C)]], srows, sem_s).wait()
            pltpu.make_async_copy(
                x_hbm.at[idx_o.at[pl.ds(c * C, C)]], orows, sem_o).wait()

        def compute(c, srows, orows):
            base = c * C
            for g in range(C // 16):
                et16 = et_all[pl.ds(base + g * 16, 16)]
                roff = et16 * W
                eids = g * 16 + lane

                def d_body(j, accs):
                    acc0, acc1 = accs
                    wcol = lane ^ j
                    sw = plsc.load_gather(srows, [eids, wcol])
                    ow = plsc.load_gather(orows, [eids, wcol])
                    rw = plsc.load_gather(rflat, [roff + wcol])
                    sv = plsc.bitcast(sw, jnp.bfloat16)
                    ov = plsc.bitcast(ow, jnp.bfloat16)
                    rv = plsc.bitcast(rw, jnp.bfloat16)
                    prod = sv * ov * rv
                    pa, pb = plsc.unpack(
                        prod, format=plsc.PackFormat.INTERLEAVED)
                    return acc0 + pa, acc1 + pb

                zero = jnp.zeros((16,), jnp.float32)
                acc0, acc1 = lax.fori_loop(
                    0, W, d_body, (zero, zero), unroll=8)
                out_all[pl.ds(base + g * 16, 16)] = acc0 + acc1

        issue(0, sr0, or0, sem_s0, sem_o0)
        issue(1, sr1, or1, sem_s1, sem_o1)

        def pair_body(i, carry):
            c0 = 2 * i
            c1 = 2 * i + 1
            wait(c0, sr0, or0, sem_s0, sem_o0)
            compute(c0, sr0, or0)

            @pl.when(c0 + 2 < NCHUNK)
            def _():
                issue(c0 + 2, sr0, or0, sem_s0, sem_o0)

            wait(c1, sr1, or1, sem_s1, sem_o1)
            compute(c1, sr1, or1)

            @pl.when(c1 + 2 < NCHUNK)
            def _():
                issue(c1 + 2, sr1, or1, sem_s1, sem_o1)

            return carry

        lax.fori_loop(0, NPAIR, pair_body, 0)

        if NCHUNK % 2 == 1:
            last = NCHUNK - 1
            wait(last, sr0, or0, sem_s0, sem_o0)
            compute(last, sr0, or0)

        pltpu.sync_copy(out_all, out_hbm.at[pl.ds(base0, EPW)])

    return dist_mult


_dist_mult_sc = _make_sc_kernel()


@jax.jit
def kernel(x, edge_index, edge_type, R_diagonal):
    ei = edge_index.astype(jnp.int32)
    et = edge_type.astype(jnp.int32)
    xp = _pack_x(x)
    return _dist_mult_sc(ei, et, xp, R_diagonal)


# speedup vs baseline: 16.9474x; 1.0070x over previous
import functools

import jax
import jax.numpy as jnp
from jax import lax
from jax.experimental import pallas as pl
from jax.experimental.pallas import tpu as pltpu
from jax.experimental.pallas import tpu_sc as plsc

N_NODES = 10000
N_EDGES = 320000
D = 128
N_REL = 64

NC = 2
NS = 16
NW = NC * NS
EPW = N_EDGES // NW
C = 80
NCHUNK = EPW // C
NPAIR = NCHUNK // 2
W = D // 2


def _pack_tc_kernel(x_ref, out_ref):
    u = pltpu.bitcast(x_ref[...], jnp.uint32)
    t = u + jnp.uint32(0x7FFF) + ((u >> 16) & jnp.uint32(1))
    lo = t[:, :W]
    hi = t[:, W:]
    packed = (hi & jnp.uint32(0xFFFF0000)) | (lo >> 16)
    out_ref[...] = pltpu.bitcast(packed, jnp.int32)


def _pack_x(x):
    return pl.pallas_call(
        _pack_tc_kernel,
        out_shape=jax.ShapeDtypeStruct((N_NODES, W), jnp.int32),
    )(x)


def _make_sc_kernel():
    mesh = plsc.VectorSubcoreMesh(core_axis_name="c", subcore_axis_name="s")

    @functools.partial(
        pl.kernel,
        out_type=jax.ShapeDtypeStruct((N_EDGES,), jnp.float32),
        mesh=mesh,
        compiler_params=pltpu.CompilerParams(
            needs_layout_passes=False, use_tc_tiling_on_sc=False),
        scratch_types=[
            pltpu.VMEM((EPW,), jnp.int32),
            pltpu.VMEM((EPW,), jnp.int32),
            pltpu.VMEM((EPW,), jnp.int32),
            pltpu.VMEM((EPW,), jnp.float32),
            pltpu.VMEM((N_REL, D), jnp.float32),
            pltpu.VMEM((N_REL * W,), jnp.int32),
            pltpu.VMEM((C, W), jnp.int32),
            pltpu.VMEM((C, W), jnp.int32),
            pltpu.VMEM((C, W), jnp.int32),
            pltpu.VMEM((C, W), jnp.int32),
            pltpu.SemaphoreType.DMA,
            pltpu.SemaphoreType.DMA,
            pltpu.SemaphoreType.DMA,
            pltpu.SemaphoreType.DMA,
        ],
    )
    def dist_mult(ei_hbm, et_hbm, x_hbm, r_hbm, out_hbm,
                  idx_s, idx_o, et_all, out_all, rtmp, rflat,
                  sr0, sr1, or0, or1, sem_s0, sem_s1, sem_o0, sem_o1):
        wid = lax.axis_index("s") * NC + lax.axis_index("c")
        base0 = wid * EPW
        pltpu.sync_copy(r_hbm, rtmp)
        pltpu.sync_copy(ei_hbm.at[0, pl.ds(base0, EPW)], idx_s)
        pltpu.sync_copy(ei_hbm.at[1, pl.ds(base0, EPW)], idx_o)
        pltpu.sync_copy(et_hbm.at[pl.ds(base0, EPW)], et_all)

        lane = lax.iota(jnp.int32, 16)

        def rpack_body(t, carry):
            for j in range(W // 16):
                a = rtmp[t, pl.ds(16 * j, 16)]
                b = rtmp[t, pl.ds(16 * j + W, 16)]
                word = plsc.bitcast(
                    plsc.pack(a, b, format=plsc.PackFormat.INTERLEAVED),
                    jnp.int32)
                rflat[pl.ds(t * W + 16 * j, 16)] = word
            return carry

        lax.fori_loop(0, N_REL, rpack_body, 0)

        def issue(c, srows, orows, sem_s, sem_o):
            pltpu.async_copy(x_hbm.at[idx_s.at[pl.ds(c * C, C)]], srows, sem_s)
            pltpu.async_copy(x_hbm.at[idx_o.at[pl.ds(c * C, C)]], orows, sem_o)

        def wait(c, srows, orows, sem_s, sem_o):
            pltpu.make_async_copy(
                x_hbm.at[idx_s.at[pl.ds(c * C, C)]], srows, sem_s).wait()
            pltpu.make_async_copy(
                x_hbm.at[idx_o.at[pl.ds(c * C, C)]], orows, sem_o).wait()

        def compute(c, srows, orows):
            base = c * C
            for g in range(C // 16):
                et16 = et_all[pl.ds(base + g * 16, 16)]
                roff = et16 * W
                eids = g * 16 + lane

                def d_body(m, accs):
                    acc0, acc1 = accs

                    def packed_prod(j):
                        wcol = lane ^ j
                        sw = plsc.load_gather(srows, [eids, wcol])
                        ow = plsc.load_gather(orows, [eids, wcol])
                        rw = plsc.load_gather(rflat, [roff + wcol])
                        sv = plsc.bitcast(sw, jnp.bfloat16)
                        ov = plsc.bitcast(ow, jnp.bfloat16)
                        rv = plsc.bitcast(rw, jnp.bfloat16)
                        return sv * ov * rv

                    ps = packed_prod(2 * m) + packed_prod(2 * m + 1)
                    pa, pb = plsc.unpack(
                        ps, format=plsc.PackFormat.INTERLEAVED)
                    return acc0 + pa, acc1 + pb

                zero = jnp.zeros((16,), jnp.float32)
                acc0, acc1 = lax.fori_loop(
                    0, W // 2, d_body, (zero, zero), unroll=8)
                out_all[pl.ds(base + g * 16, 16)] = acc0 + acc1

        issue(0, sr0, or0, sem_s0, sem_o0)
        issue(1, sr1, or1, sem_s1, sem_o1)

        def pair_body(i, carry):
            c0 = 2 * i
            c1 = 2 * i + 1
            wait(c0, sr0, or0, sem_s0, sem_o0)
            compute(c0, sr0, or0)

            @pl.when(c0 + 2 < NCHUNK)
            def _():
                issue(c0 + 2, sr0, or0, sem_s0, sem_o0)

            wait(c1, sr1, or1, sem_s1, sem_o1)
            compute(c1, sr1, or1)

            @pl.when(c1 + 2 < NCHUNK)
            def _():
                issue(c1 + 2, sr1, or1, sem_s1, sem_o1)

            return carry

        lax.fori_loop(0, NPAIR, pair_body, 0)

        if NCHUNK % 2 == 1:
            last = NCHUNK - 1
            wait(last, sr0, or0, sem_s0, sem_o0)
            compute(last, sr0, or0)

        pltpu.sync_copy(out_all, out_hbm.at[pl.ds(base0, EPW)])

    return dist_mult


_dist_mult_sc = _make_sc_kernel()


@jax.jit
def kernel(x, edge_index, edge_type, R_diagonal):
    ei = edge_index.astype(jnp.int32)
    et = edge_type.astype(jnp.int32)
    xp = _pack_x(x)
    return _dist_mult_sc(ei, et, xp, R_diagonal)
